# baseline (speedup 1.0000x reference)
import functools

import jax
import jax.numpy as jnp
from jax import lax
from jax.experimental import pallas as pl
from jax.experimental.pallas import tpu as pltpu
from jax.experimental.pallas import tpu_sc as plsc

E = 16
K = 4
H = 1024
FF = 1024
B = 16
ALPHA = 1.702
LIMIT = 7.0
EPS = 1e-5
FF2 = FF // 2
H2 = H // 2


def _prep_body(x_ref, ns_ref, gw_ref, gb_ref, t_ref, g_ref):
    xv = x_ref[...]
    t = xv * lax.rsqrt(jnp.mean(xv * xv, axis=-1, keepdims=True) + EPS)
    t = t * ns_ref[...]
    t_ref[...] = t
    g_ref[...] = jax.lax.dot_general(
        t, gw_ref[...], (((1,), (1,)), ((), ())),
        preferred_element_type=jnp.float32) + gb_ref[...]


def _make_router():
    mesh = plsc.VectorSubcoreMesh(core_axis_name="c", subcore_axis_name="s")

    @functools.partial(
        pl.kernel, mesh=mesh,
        out_type=jax.ShapeDtypeStruct((B, E), jnp.float32),
        scratch_types=[pltpu.VMEM((E,), jnp.float32),
                       pltpu.VMEM((E,), jnp.float32)],
    )
    def _route(g_hbm, w_hbm, g_v, w_v):
        cid = lax.axis_index("c")
        sid = lax.axis_index("s")
        wid = sid * 2 + cid

        lanes = lax.iota(jnp.int32, E)
        rots = [jnp.bitwise_and(lanes + sh, E - 1) for sh in (8, 4, 2, 1)]

        gdn = lax.GatherDimensionNumbers(offset_dims=(),
                                         collapsed_slice_dims=(0,),
                                         start_index_map=(0,))

        def permute(v, idx):
            return lax.gather(v, idx[:, None], gdn, slice_sizes=(1,),
                              mode=lax.GatherScatterMode.PROMISE_IN_BOUNDS)

        def allred(v, op):
            for idx in rots:
                v = op(v, permute(v, idx))
            return v

        @pl.when(wid < B)
        def _():
            pltpu.sync_copy(g_hbm.at[wid], g_v)
            g = g_v[...]
            g_work = g
            for _k in range(K):
                m = allred(g_work, jnp.maximum)
                first = allred(jnp.where(g_work == m, lanes, E), jnp.minimum)
                g_work = jnp.where(lanes == first, -1e30, g_work)
            vals = jnp.where(g_work < -1e29, g, -1e30)
            mx = allred(vals, jnp.maximum)
            ex = jnp.exp(vals - mx)
            w_v[...] = ex / allred(ex, jnp.add)
            pltpu.sync_copy(w_v, w_hbm.at[wid])

    return _route


_router = _make_router()


def _moe_body(x_ref, t_ref, w_ref, m1a_ref, m1b_ref, b1g_ref, b1l_ref,
              m2a_ref, m2b_ref, b2_ref, out_ref):
    e = pl.program_id(0)

    @pl.when(e == 0)
    def _init():
        out_ref[...] = x_ref[...]

    t = t_ref[...].astype(jnp.bfloat16)

    def half(m1_ref):
        m1 = m1_ref[0].astype(jnp.bfloat16)
        hg = jax.lax.dot_general(t, m1[:, :H], (((1,), (1,)), ((), ())),
                                 preferred_element_type=jnp.float32)
        hl = jax.lax.dot_general(t, m1[:, H:], (((1,), (1,)), ((), ())),
                                 preferred_element_type=jnp.float32)
        return hg, hl

    hga, hla = half(m1a_ref)
    hgb, hlb = half(m1b_ref)
    hg = jnp.concatenate([hga, hgb], axis=1) + b1g_ref[0]
    hl = jnp.concatenate([hla, hlb], axis=1) + b1l_ref[0]
    hg = jnp.minimum(hg, LIMIT)
    hl = jnp.clip(hl, -LIMIT, LIMIT)
    t2 = (hg * jax.nn.sigmoid(ALPHA * hg) * (hl + 1.0)).astype(jnp.bfloat16)
    t3a = jax.lax.dot_general(t2, m2a_ref[0].astype(jnp.bfloat16),
                              (((1,), (1,)), ((), ())),
                              preferred_element_type=jnp.float32)
    t3b = jax.lax.dot_general(t2, m2b_ref[0].astype(jnp.bfloat16),
                              (((1,), (1,)), ((), ())),
                              preferred_element_type=jnp.float32)
    t3 = jnp.concatenate([t3a, t3b], axis=1) + b2_ref[0]
    lane = jax.lax.broadcasted_iota(jnp.int32, (B, E), 1)
    w_e = jnp.sum(jnp.where(lane == e, w_ref[...], 0.0), axis=1,
                  keepdims=True)
    out_ref[...] += t3 * w_e


def kernel(x, norm_scale, gate_w, gate_b, mlp1_w, mlp1_b, mlp2_w, mlp2_b):
    ns = norm_scale.reshape(1, H)
    gb = gate_b.reshape(1, E)
    t, g = pl.pallas_call(
        _prep_body,
        in_specs=[pl.BlockSpec((B, H), lambda: (0, 0)),
                  pl.BlockSpec((1, H), lambda: (0, 0)),
                  pl.BlockSpec((E, H), lambda: (0, 0)),
                  pl.BlockSpec((1, E), lambda: (0, 0))],
        out_specs=[pl.BlockSpec((B, H), lambda: (0, 0)),
                   pl.BlockSpec((B, E), lambda: (0, 0))],
        out_shape=[jax.ShapeDtypeStruct((B, H), jnp.float32),
                   jax.ShapeDtypeStruct((B, E), jnp.float32)],
    )(x, ns, gate_w, gb)

    w = _router(g)

    m1r = mlp1_w.reshape(E, FF, 2 * H)
    b1 = mlp1_b.reshape(E, 1, FF, 2)
    b1g = b1[..., 0]
    b1l = b1[..., 1]
    b2 = mlp2_b.reshape(E, 1, H)
    out = pl.pallas_call(
        _moe_body,
        grid=(E,),
        in_specs=[
            pl.BlockSpec((B, H), lambda e: (0, 0)),
            pl.BlockSpec((B, H), lambda e: (0, 0)),
            pl.BlockSpec((B, E), lambda e: (0, 0)),
            pl.BlockSpec((1, FF2, 2 * H), lambda e: (e, 0, 0)),
            pl.BlockSpec((1, FF2, 2 * H), lambda e: (e, 1, 0)),
            pl.BlockSpec((1, 1, FF), lambda e: (e, 0, 0)),
            pl.BlockSpec((1, 1, FF), lambda e: (e, 0, 0)),
            pl.BlockSpec((1, H2, FF), lambda e: (e, 0, 0)),
            pl.BlockSpec((1, H2, FF), lambda e: (e, 1, 0)),
            pl.BlockSpec((1, 1, H), lambda e: (e, 0, 0)),
        ],
        out_specs=pl.BlockSpec((B, H), lambda e: (0, 0)),
        out_shape=jax.ShapeDtypeStruct((B, H), jnp.float32),
        compiler_params=pltpu.CompilerParams(
            dimension_semantics=("arbitrary",)),
    )(x, t, w, m1r, m1r, b1g, b1l, mlp2_w, mlp2_w, b2)
    return out
